# baseline (speedup 1.0000x reference)
import functools

import jax
import jax.numpy as jnp
from jax import lax
from jax.experimental import pallas as pl
from jax.experimental.pallas import tpu as pltpu, tpu_sc as plsc

NUM_NODES = 1000000
D_MSG = 64
BATCH = 16384

NC = 2
NS = 16
NW = NC * NS
B_PER_W = BATCH // NW
B_PER_S = BATCH // NS
IDX_CHUNK = 128
M_CHUNKS = B_PER_W // IDX_CHUNK
T_CHUNKS = B_PER_S // IDX_CHUNK
L = 16


def _sc_kernel_body(node_ids, prev_ts, msg_t, msg_ts, out_msgs, out_ts,
                    idx_m, idx_t, fbuf, rows_v, ts_v, prev_v, acc_ref,
                    allcnt_v, shared_cnt, sem_m, sem_t):
    cid = lax.axis_index("c")
    sid = lax.axis_index("s")
    wid = sid * NC + cid
    base = wid * B_PER_W
    tbase = sid * B_PER_S

    for j in range(M_CHUNKS):
        pltpu.sync_copy(node_ids.at[pl.ds(base + j * IDX_CHUNK, IDX_CHUNK)],
                        idx_m.at[j])
    for j in range(T_CHUNKS):
        pltpu.sync_copy(node_ids.at[pl.ds(tbase + j * IDX_CHUNK, IDX_CHUNK)],
                        idx_t.at[j])

    ts_copies = [
        pltpu.async_copy(msg_ts.at[idx_t.at[j]],
                         ts_v.at[pl.ds(j * IDX_CHUNK, IDX_CHUNK)], sem_t)
        for j in range(T_CHUNKS)
    ]
    pltpu.sync_copy(prev_ts.at[pl.ds(tbase, B_PER_S)], prev_v)

    gathers = [
        pltpu.async_copy(msg_t.at[f].at[idx_m.at[jc]],
                         fbuf.at[f, pl.ds(jc * IDX_CHUNK, IDX_CHUNK)],
                         sem_m)
        for f in range(D_MSG)
        for jc in range(M_CHUNKS)
    ]
    for c in gathers:
        c.wait()

    iota = lax.iota(jnp.int32, L) * D_MSG

    def transpose_g(g, carry):
        kvec = iota + g * (L * D_MSG)
        for f in range(D_MSG):
            v = fbuf[f, pl.ds(g * L, L)]
            plsc.store_scatter(rows_v, [kvec + f], v)
        return carry

    lax.fori_loop(0, B_PER_W // L, transpose_g, 0)

    for c in ts_copies:
        c.wait()

    one = jnp.full((L,), 1.0, jnp.float32)
    zero = jnp.full((L,), 0.0, jnp.float32)
    acc = zero
    for j in range(B_PER_S // L):
        sl = pl.ds(j * L, L)
        acc = acc + jnp.where(prev_v[sl] > ts_v[sl], one, zero)
    acc_ref[...] = acc

    pltpu.sync_copy(acc_ref, shared_cnt.at[sid])
    plsc.subcore_barrier()
    pltpu.sync_copy(shared_cnt, allcnt_v)
    total_vec = zero
    for i in range(NS):
        total_vec = total_vec + allcnt_v[i]
    total = total_vec[0]
    for l in range(1, L):
        total = total + total_vec[l]
    invalid = total > 0.0

    @pl.when(invalid)
    def _poison():
        nan_vec = jnp.full((L,), jnp.nan, jnp.float32)

        def body(i, carry):
            rows_v[pl.ds(i * L, L)] = nan_vec
            return carry

        lax.fori_loop(0, (B_PER_W * D_MSG) // L, body, 0)
        for j in range(B_PER_S // L):
            ts_v[pl.ds(j * L, L)] = nan_vec

    pltpu.sync_copy(rows_v, out_msgs.at[pl.ds(base * D_MSG, B_PER_W * D_MSG)])

    @pl.when(cid == 0)
    def _store_ts():
        pltpu.sync_copy(ts_v, out_ts.at[pl.ds(tbase, B_PER_S)])


@jax.jit
def _last_message_gather(node_ids, prev_ts, msg_store, msg_ts):
    msg_t = msg_store.T
    mesh = plsc.VectorSubcoreMesh(core_axis_name="c", subcore_axis_name="s")
    kfn = functools.partial(
        pl.kernel,
        out_type=(
            jax.ShapeDtypeStruct((BATCH * D_MSG,), jnp.float32),
            jax.ShapeDtypeStruct((BATCH,), jnp.float32),
        ),
        mesh=mesh,
        compiler_params=pltpu.CompilerParams(
            needs_layout_passes=False, use_tc_tiling_on_sc=False),
        scratch_types=[
            pltpu.VMEM((M_CHUNKS, IDX_CHUNK), jnp.int32),
            pltpu.VMEM((T_CHUNKS, IDX_CHUNK), jnp.int32),
            pltpu.VMEM((D_MSG, B_PER_W), jnp.float32),
            pltpu.VMEM((B_PER_W * D_MSG,), jnp.float32),
            pltpu.VMEM((B_PER_S,), jnp.float32),
            pltpu.VMEM((B_PER_S,), jnp.float32),
            pltpu.VMEM((L,), jnp.float32),
            pltpu.VMEM((NS, L), jnp.float32),
            pltpu.VMEM_SHARED((NS, L), jnp.float32),
            pltpu.SemaphoreType.DMA,
            pltpu.SemaphoreType.DMA,
        ],
    )(_sc_kernel_body)
    out_flat, ts = kfn(node_ids, prev_ts, msg_t, msg_ts)
    return jnp.reshape(out_flat, (BATCH, D_MSG)), ts


def kernel(node_ids, prev_ts, msg_store, msg_ts):
    return _last_message_gather(node_ids, prev_ts, msg_store, msg_ts)


# speedup vs baseline: 13.7187x; 13.7187x over previous
import functools

import jax
import jax.numpy as jnp
from jax import lax
from jax.experimental import pallas as pl
from jax.experimental.pallas import tpu as pltpu, tpu_sc as plsc

NUM_NODES = 1000000
D_MSG = 64
BATCH = 16384

NC = 2
NS = 16
NW = NC * NS
B_PER_W = BATCH // NW
B_PER_S = BATCH // NS
IDX_CHUNK = 128
M_CHUNKS = B_PER_W // IDX_CHUNK
T_CHUNKS = B_PER_S // IDX_CHUNK
L = 16


def _sc_kernel_body(node_ids, prev_ts, msg_store, msg_ts, out_msgs, out_ts,
                    idx_m, idx_t, rows_v, ts_v, prev_v, acc_ref, allcnt_v,
                    shared_cnt, sem_m, sem_t):
    cid = lax.axis_index("c")
    sid = lax.axis_index("s")
    wid = sid * NC + cid
    base = wid * B_PER_W
    tbase = sid * B_PER_S

    pltpu.sync_copy(node_ids.at[pl.ds(base, B_PER_W)], idx_m)
    for j in range(T_CHUNKS):
        pltpu.sync_copy(node_ids.at[pl.ds(tbase + j * IDX_CHUNK, IDX_CHUNK)],
                        idx_t.at[j])

    def issue_rows(g, carry):
        vec = idx_m[pl.ds(g * L, L)]
        for u in range(L):
            r = vec[u]
            pltpu.async_copy(msg_store.at[pl.ds(r, 1)],
                             rows_v.at[pl.ds(g * L + u, 1)], sem_m)
        return carry

    lax.fori_loop(0, B_PER_W // L, issue_rows, 0)
    row_copies = [
        pltpu.make_async_copy(msg_store.at[pl.ds(0, B_PER_W)], rows_v, sem_m)
    ]
    ts_copies = [
        pltpu.async_copy(msg_ts.at[idx_t.at[j]],
                         ts_v.at[pl.ds(j * IDX_CHUNK, IDX_CHUNK)], sem_t)
        for j in range(T_CHUNKS)
    ]
    pltpu.sync_copy(prev_ts.at[pl.ds(tbase, B_PER_S)], prev_v)
    for c in ts_copies:
        c.wait()

    one = jnp.full((L,), 1.0, jnp.float32)
    zero = jnp.full((L,), 0.0, jnp.float32)
    acc = zero
    for j in range(B_PER_S // L):
        sl = pl.ds(j * L, L)
        acc = acc + jnp.where(prev_v[sl] > ts_v[sl], one, zero)
    acc_ref[...] = acc

    pltpu.sync_copy(acc_ref, shared_cnt.at[sid])
    plsc.subcore_barrier()
    pltpu.sync_copy(shared_cnt, allcnt_v)
    total_vec = zero
    for i in range(NS):
        total_vec = total_vec + allcnt_v[i]
    total = total_vec[0]
    for l in range(1, L):
        total = total + total_vec[l]
    invalid = total > 0.0

    for c in row_copies:
        c.wait()

    @pl.when(invalid)
    def _poison():
        nan_vec = jnp.full((L,), jnp.nan, jnp.float32)

        def body(i, carry):
            for j in range(D_MSG // L):
                rows_v[i, pl.ds(j * L, L)] = nan_vec
            return carry

        lax.fori_loop(0, B_PER_W, body, 0)
        for j in range(B_PER_S // L):
            ts_v[pl.ds(j * L, L)] = nan_vec

    pltpu.sync_copy(rows_v, out_msgs.at[pl.ds(base, B_PER_W)])

    @pl.when(cid == 0)
    def _store_ts():
        pltpu.sync_copy(ts_v, out_ts.at[pl.ds(tbase, B_PER_S)])


@jax.jit
def _last_message_gather(node_ids, prev_ts, msg_store, msg_ts):
    mesh = plsc.VectorSubcoreMesh(core_axis_name="c", subcore_axis_name="s")
    kfn = functools.partial(
        pl.kernel,
        out_type=(
            jax.ShapeDtypeStruct((BATCH, D_MSG), jnp.float32),
            jax.ShapeDtypeStruct((BATCH,), jnp.float32),
        ),
        mesh=mesh,
        scratch_types=[
            pltpu.VMEM((B_PER_W,), jnp.int32),
            pltpu.VMEM((T_CHUNKS, IDX_CHUNK), jnp.int32),
            pltpu.VMEM((B_PER_W, D_MSG), jnp.float32),
            pltpu.VMEM((B_PER_S,), jnp.float32),
            pltpu.VMEM((B_PER_S,), jnp.float32),
            pltpu.VMEM((L,), jnp.float32),
            pltpu.VMEM((NS, L), jnp.float32),
            pltpu.VMEM_SHARED((NS, L), jnp.float32),
            pltpu.SemaphoreType.DMA,
            pltpu.SemaphoreType.DMA,
        ],
    )(_sc_kernel_body)
    return kfn(node_ids, prev_ts, msg_store, msg_ts)


def kernel(node_ids, prev_ts, msg_store, msg_ts):
    return _last_message_gather(node_ids, prev_ts, msg_store, msg_ts)


# speedup vs baseline: 17.2837x; 1.2599x over previous
import functools

import jax
import jax.numpy as jnp
from jax import lax
from jax.experimental import pallas as pl
from jax.experimental.pallas import tpu as pltpu, tpu_sc as plsc

NUM_NODES = 1000000
D_MSG = 64
BATCH = 16384

NC = 2
NS = 16
NW = NC * NS
B_PER_W = BATCH // NW
B_PER_S = BATCH // NS
IDX_CHUNK = 128
T_CHUNKS = B_PER_S // IDX_CHUNK
L = 16
WV = 4
LANES = 128


def _sc_kernel_body(node_ids, prev_ts, msg_t, msg_ts, out_flat, out_ts,
                    idx_m, idx_t, buf_a, buf_b, colbuf, ts_v, prev_v,
                    acc_ref, allcnt_v, shared_cnt, sem_a, sem_b, sem_t,
                    sem_o):
    cid = lax.axis_index("c")
    sid = lax.axis_index("s")
    wid = sid * NC + cid
    base = wid * B_PER_W
    tbase = sid * B_PER_S

    pltpu.sync_copy(node_ids.at[pl.ds(base, B_PER_W)],
                    idx_m.at[pl.ds(0, B_PER_W)])
    for j in range(T_CHUNKS):
        pltpu.sync_copy(node_ids.at[pl.ds(tbase + j * IDX_CHUNK, IDX_CHUNK)],
                        idx_t.at[j])

    ts_copies = [
        pltpu.async_copy(msg_ts.at[idx_t.at[j]],
                         ts_v.at[pl.ds(j * IDX_CHUNK, IDX_CHUNK)], sem_t)
        for j in range(T_CHUNKS)
    ]
    pltpu.sync_copy(prev_ts.at[pl.ds(tbase, B_PER_S)], prev_v)
    for c in ts_copies:
        c.wait()

    one = jnp.full((L,), 1.0, jnp.float32)
    zero = jnp.full((L,), 0.0, jnp.float32)
    acc = zero
    for j in range(B_PER_S // L):
        sl = pl.ds(j * L, L)
        acc = acc + jnp.where(prev_v[sl] > ts_v[sl], one, zero)
    acc_ref[...] = acc
    pltpu.sync_copy(acc_ref, shared_cnt.at[sid])
    plsc.subcore_barrier()
    pltpu.sync_copy(shared_cnt, allcnt_v)
    total_vec = zero
    for i in range(NS):
        total_vec = total_vec + allcnt_v[i]
    total = total_vec[0]
    for l in range(1, L):
        total = total + total_vec[l]
    nan_vec = jnp.where(total > 0.0, jnp.full((L,), jnp.nan, jnp.float32),
                        zero)

    for j in range(B_PER_S // L):
        sl = pl.ds(j * L, L)
        ts_v[sl] = ts_v[sl] + nan_vec

    @pl.when(cid == 0)
    def _store_ts():
        pltpu.sync_copy(ts_v, out_ts.at[pl.ds(tbase, B_PER_S)])

    seven_bits = jnp.full((L,), 7, jnp.int32)
    del seven_bits
    cmask = jnp.full((L,), LANES - 1, jnp.int32)
    fvecs = [lax.iota(jnp.int32, L) + c * L for c in range(D_MSG // L)]

    def body(t, carry):
        vec = idx_m[pl.ds(t * 8, L)]
        coff = vec - lax.bitwise_and(vec, cmask)
        lane = lax.bitwise_and(vec, cmask)

        def fire(u, buf, sem):
            off = pl.multiple_of(coff[u], LANES)
            return pltpu.async_copy(
                msg_t.at[pl.ds(0, D_MSG), pl.ds(off, LANES)],
                buf.at[u % WV], sem)
        copies_a = [fire(u, buf_a, sem_a) for u in range(0, WV)]
        copies_b = [fire(u, buf_b, sem_b) for u in range(WV, 2 * WV)]

        def extract(u, buf):
            lane_u = zero.astype(jnp.int32) + lane[u]
            usplat = zero.astype(jnp.int32) + (u % WV)
            for c in range(D_MSG // L):
                v = plsc.load_gather(buf, [usplat, fvecs[c], lane_u])
                colbuf[u, pl.ds(c * L, L)] = v + nan_vec

        for cpy in copies_a:
            cpy.wait()
        for u in range(0, WV):
            extract(u, buf_a)
        for cpy in copies_b:
            cpy.wait()
        for u in range(WV, 2 * WV):
            extract(u, buf_b)

        out_cps = [
            pltpu.async_copy(
                colbuf.at[u],
                out_flat.at[pl.ds((base + t * 8 + u) * D_MSG, D_MSG)],
                sem_o)
            for u in range(8)
        ]
        for cpy in out_cps:
            cpy.wait()
        return carry

    lax.fori_loop(0, B_PER_W // 8, body, 0)


@jax.jit
def _last_message_gather(node_ids, prev_ts, msg_store, msg_ts):
    msg_t = msg_store.T
    mesh = plsc.VectorSubcoreMesh(core_axis_name="c", subcore_axis_name="s")
    kfn = functools.partial(
        pl.kernel,
        out_type=(
            jax.ShapeDtypeStruct((BATCH * D_MSG,), jnp.float32),
            jax.ShapeDtypeStruct((BATCH,), jnp.float32),
        ),
        mesh=mesh,
        compiler_params=pltpu.CompilerParams(needs_layout_passes=False),
        scratch_types=[
            pltpu.VMEM((B_PER_W + 8,), jnp.int32),
            pltpu.VMEM((T_CHUNKS, IDX_CHUNK), jnp.int32),
            pltpu.VMEM((WV, D_MSG, LANES), jnp.float32),
            pltpu.VMEM((WV, D_MSG, LANES), jnp.float32),
            pltpu.VMEM((8, D_MSG), jnp.float32),
            pltpu.VMEM((B_PER_S,), jnp.float32),
            pltpu.VMEM((B_PER_S,), jnp.float32),
            pltpu.VMEM((L,), jnp.float32),
            pltpu.VMEM((NS, L), jnp.float32),
            pltpu.VMEM_SHARED((NS, L), jnp.float32),
            pltpu.SemaphoreType.DMA,
            pltpu.SemaphoreType.DMA,
            pltpu.SemaphoreType.DMA,
            pltpu.SemaphoreType.DMA,
        ],
    )(_sc_kernel_body)
    out_flat, ts = kfn(node_ids, prev_ts, msg_t, msg_ts)
    return jnp.reshape(out_flat, (BATCH, D_MSG)), ts


def kernel(node_ids, prev_ts, msg_store, msg_ts):
    return _last_message_gather(node_ids, prev_ts, msg_store, msg_ts)


# speedup vs baseline: 17.5582x; 1.0159x over previous
import functools

import jax
import jax.numpy as jnp
from jax import lax
from jax.experimental import pallas as pl
from jax.experimental.pallas import tpu as pltpu, tpu_sc as plsc

NUM_NODES = 1000000
D_MSG = 64
BATCH = 16384

NC = 2
NS = 16
NW = NC * NS
B_PER_W = BATCH // NW
B_PER_S = BATCH // NS
IDX_CHUNK = 128
T_CHUNKS = B_PER_S // IDX_CHUNK
L = 16
WV = 4
LANES = 128


def _sc_kernel_body(node_ids, prev_ts, msg_t, msg_ts, out_flat, out_ts,
                    idx_m, idx_t, buf_a, buf_b, colbuf, ts_v, prev_v,
                    acc_ref, allcnt_v, shared_cnt, sem_a, sem_b, sem_t,
                    sem_o):
    cid = lax.axis_index("c")
    sid = lax.axis_index("s")
    wid = sid * NC + cid
    base = wid * B_PER_W
    tbase = sid * B_PER_S

    pltpu.sync_copy(node_ids.at[pl.ds(base, B_PER_W)],
                    idx_m.at[pl.ds(0, B_PER_W)])
    for j in range(T_CHUNKS):
        pltpu.sync_copy(node_ids.at[pl.ds(tbase + j * IDX_CHUNK, IDX_CHUNK)],
                        idx_t.at[j])

    ts_copies = [
        pltpu.async_copy(msg_ts.at[idx_t.at[j]],
                         ts_v.at[pl.ds(j * IDX_CHUNK, IDX_CHUNK)], sem_t)
        for j in range(T_CHUNKS)
    ]
    pltpu.sync_copy(prev_ts.at[pl.ds(tbase, B_PER_S)], prev_v)
    for c in ts_copies:
        c.wait()

    one = jnp.full((L,), 1.0, jnp.float32)
    zero = jnp.full((L,), 0.0, jnp.float32)
    acc = zero
    for j in range(B_PER_S // L):
        sl = pl.ds(j * L, L)
        acc = acc + jnp.where(prev_v[sl] > ts_v[sl], one, zero)
    acc_ref[...] = acc
    pltpu.sync_copy(acc_ref, shared_cnt.at[sid])
    plsc.subcore_barrier()
    pltpu.sync_copy(shared_cnt, allcnt_v)
    total_vec = zero
    for i in range(NS):
        total_vec = total_vec + allcnt_v[i]
    total = total_vec[0]
    for l in range(1, L):
        total = total + total_vec[l]
    nan_vec = jnp.where(total > 0.0, jnp.full((L,), jnp.nan, jnp.float32),
                        zero)

    for j in range(B_PER_S // L):
        sl = pl.ds(j * L, L)
        ts_v[sl] = ts_v[sl] + nan_vec

    @pl.when(cid == 0)
    def _store_ts():
        pltpu.sync_copy(ts_v, out_ts.at[pl.ds(tbase, B_PER_S)])

    seven_bits = jnp.full((L,), 7, jnp.int32)
    del seven_bits
    cmask = jnp.full((L,), LANES - 1, jnp.int32)
    fvecs = [lax.iota(jnp.int32, L) + c * L for c in range(D_MSG // L)]

    def body(t, carry):
        vec = idx_m[pl.ds(t * 8, L)]
        coff = vec - lax.bitwise_and(vec, cmask)
        lane = lax.bitwise_and(vec, cmask)

        def fire(u, buf, sem):
            off = pl.multiple_of(coff[u], LANES)
            return pltpu.async_copy(
                msg_t.at[pl.ds(0, D_MSG), pl.ds(off, LANES)],
                buf.at[u % WV], sem)
        copies_a = [fire(u, buf_a, sem_a) for u in range(0, WV)]
        copies_b = [fire(u, buf_b, sem_b) for u in range(WV, 2 * WV)]

        def extract(u, buf):
            lane_u = zero.astype(jnp.int32) + lane[u]
            usplat = zero.astype(jnp.int32) + (u % WV)
            for c in range(D_MSG // L):
                v = plsc.load_gather(buf, [usplat, fvecs[c], lane_u])
                colbuf[pl.ds((t * 8 + u) * D_MSG + c * L, L)] = v + nan_vec

        for cpy in copies_a:
            cpy.wait()
        for u in range(0, WV):
            extract(u, buf_a)
        for cpy in copies_b:
            cpy.wait()
        for u in range(WV, 2 * WV):
            extract(u, buf_b)

        pltpu.async_copy(
            colbuf.at[pl.ds((t * 8) * D_MSG, 8 * D_MSG)],
            out_flat.at[pl.ds((base + t * 8) * D_MSG, 8 * D_MSG)],
            sem_o)
        return carry

    lax.fori_loop(0, B_PER_W // 8, body, 0)
    pltpu.make_async_copy(
        out_flat.at[pl.ds(0, B_PER_W * D_MSG)], colbuf, sem_o).wait()


@jax.jit
def _last_message_gather(node_ids, prev_ts, msg_store, msg_ts):
    msg_t = msg_store.T
    mesh = plsc.VectorSubcoreMesh(core_axis_name="c", subcore_axis_name="s")
    kfn = functools.partial(
        pl.kernel,
        out_type=(
            jax.ShapeDtypeStruct((BATCH * D_MSG,), jnp.float32),
            jax.ShapeDtypeStruct((BATCH,), jnp.float32),
        ),
        mesh=mesh,
        compiler_params=pltpu.CompilerParams(needs_layout_passes=False),
        scratch_types=[
            pltpu.VMEM((B_PER_W + 8,), jnp.int32),
            pltpu.VMEM((T_CHUNKS, IDX_CHUNK), jnp.int32),
            pltpu.VMEM((WV, D_MSG, LANES), jnp.float32),
            pltpu.VMEM((WV, D_MSG, LANES), jnp.float32),
            pltpu.VMEM((B_PER_W * D_MSG,), jnp.float32),
            pltpu.VMEM((B_PER_S,), jnp.float32),
            pltpu.VMEM((B_PER_S,), jnp.float32),
            pltpu.VMEM((L,), jnp.float32),
            pltpu.VMEM((NS, L), jnp.float32),
            pltpu.VMEM_SHARED((NS, L), jnp.float32),
            pltpu.SemaphoreType.DMA,
            pltpu.SemaphoreType.DMA,
            pltpu.SemaphoreType.DMA,
            pltpu.SemaphoreType.DMA,
        ],
    )(_sc_kernel_body)
    out_flat, ts = kfn(node_ids, prev_ts, msg_t, msg_ts)
    return jnp.reshape(out_flat, (BATCH, D_MSG)), ts


def kernel(node_ids, prev_ts, msg_store, msg_ts):
    return _last_message_gather(node_ids, prev_ts, msg_store, msg_ts)


# speedup vs baseline: 17.7321x; 1.0099x over previous
import functools

import jax
import jax.numpy as jnp
from jax import lax
from jax.experimental import pallas as pl
from jax.experimental.pallas import tpu as pltpu, tpu_sc as plsc

NUM_NODES = 1000000
D_MSG = 64
BATCH = 16384

NC = 2
NS = 16
NW = NC * NS
B_PER_W = BATCH // NW
B_PER_S = BATCH // NS
IDX_CHUNK = 128
T_CHUNKS = B_PER_S // IDX_CHUNK
L = 16
WV = 4
LANES = 128


def _sc_kernel_body(node_ids, prev_ts, msg_t, msg_ts, out_flat, out_ts,
                    idx_m, idx_t, buf_a, buf_b, colbuf, ts_v, prev_v,
                    acc_ref, allcnt_v, shared_cnt, sem_a, sem_b, sem_t,
                    sem_o):
    cid = lax.axis_index("c")
    sid = lax.axis_index("s")
    wid = sid * NC + cid
    base = wid * B_PER_W
    tbase = sid * B_PER_S

    pltpu.sync_copy(node_ids.at[pl.ds(base, B_PER_W)],
                    idx_m.at[pl.ds(0, B_PER_W)])
    for j in range(T_CHUNKS):
        pltpu.sync_copy(node_ids.at[pl.ds(tbase + j * IDX_CHUNK, IDX_CHUNK)],
                        idx_t.at[j])

    ts_copies = [
        pltpu.async_copy(msg_ts.at[idx_t.at[j]],
                         ts_v.at[pl.ds(j * IDX_CHUNK, IDX_CHUNK)], sem_t)
        for j in range(T_CHUNKS)
    ]
    pltpu.sync_copy(prev_ts.at[pl.ds(tbase, B_PER_S)], prev_v)
    zero = jnp.full((L,), 0.0, jnp.float32)

    seven_bits = jnp.full((L,), 7, jnp.int32)
    del seven_bits
    cmask = jnp.full((L,), LANES - 1, jnp.int32)
    fvecs = [lax.iota(jnp.int32, L) + c * L for c in range(D_MSG // L)]

    def body(t, carry):
        vec = idx_m[pl.ds(t * 8, L)]
        coff = vec - lax.bitwise_and(vec, cmask)
        lane = lax.bitwise_and(vec, cmask)

        def fire(u, buf, sem):
            off = pl.multiple_of(coff[u], LANES)
            return pltpu.async_copy(
                msg_t.at[pl.ds(0, D_MSG), pl.ds(off, LANES)],
                buf.at[u % WV], sem)
        copies_a = [fire(u, buf_a, sem_a) for u in range(0, WV)]
        copies_b = [fire(u, buf_b, sem_b) for u in range(WV, 2 * WV)]

        def extract(u, buf):
            lane_u = zero.astype(jnp.int32) + lane[u]
            usplat = zero.astype(jnp.int32) + (u % WV)
            for c in range(D_MSG // L):
                v = plsc.load_gather(buf, [usplat, fvecs[c], lane_u])
                colbuf[pl.ds((t * 8 + u) * D_MSG + c * L, L)] = v

        for cpy in copies_a:
            cpy.wait()
        for u in range(0, WV):
            extract(u, buf_a)
        for cpy in copies_b:
            cpy.wait()
        for u in range(WV, 2 * WV):
            extract(u, buf_b)

        pltpu.async_copy(
            colbuf.at[pl.ds((t * 8) * D_MSG, 8 * D_MSG)],
            out_flat.at[pl.ds((base + t * 8) * D_MSG, 8 * D_MSG)],
            sem_o)
        return carry

    lax.fori_loop(0, B_PER_W // 8, body, 0)

    for c in ts_copies:
        c.wait()
    one = jnp.full((L,), 1.0, jnp.float32)
    acc = zero
    for j in range(B_PER_S // L):
        sl = pl.ds(j * L, L)
        acc = acc + jnp.where(prev_v[sl] > ts_v[sl], one, zero)
    acc_ref[...] = acc
    pltpu.sync_copy(acc_ref, shared_cnt.at[sid])
    plsc.subcore_barrier()
    pltpu.sync_copy(shared_cnt, allcnt_v)
    total_vec = zero
    for i in range(NS):
        total_vec = total_vec + allcnt_v[i]
    total = total_vec[0]
    for l in range(1, L):
        total = total + total_vec[l]
    invalid = total > 0.0
    nan_vec = jnp.where(invalid, jnp.full((L,), jnp.nan, jnp.float32), zero)

    for j in range(B_PER_S // L):
        sl = pl.ds(j * L, L)
        ts_v[sl] = ts_v[sl] + nan_vec

    @pl.when(cid == 0)
    def _store_ts():
        pltpu.sync_copy(ts_v, out_ts.at[pl.ds(tbase, B_PER_S)])

    pltpu.make_async_copy(
        out_flat.at[pl.ds(0, B_PER_W * D_MSG)], colbuf, sem_o).wait()

    @pl.when(invalid)
    def _poison():
        nanv = jnp.full((L,), jnp.nan, jnp.float32)

        def fill(i, carry):
            colbuf[pl.ds(i * L, L)] = nanv
            return carry

        lax.fori_loop(0, (B_PER_W * D_MSG) // L, fill, 0)
        pltpu.sync_copy(colbuf,
                        out_flat.at[pl.ds(base * D_MSG, B_PER_W * D_MSG)])


@jax.jit
def _last_message_gather(node_ids, prev_ts, msg_store, msg_ts):
    msg_t = msg_store.T
    mesh = plsc.VectorSubcoreMesh(core_axis_name="c", subcore_axis_name="s")
    kfn = functools.partial(
        pl.kernel,
        out_type=(
            jax.ShapeDtypeStruct((BATCH * D_MSG,), jnp.float32),
            jax.ShapeDtypeStruct((BATCH,), jnp.float32),
        ),
        mesh=mesh,
        compiler_params=pltpu.CompilerParams(needs_layout_passes=False),
        scratch_types=[
            pltpu.VMEM((B_PER_W + 8,), jnp.int32),
            pltpu.VMEM((T_CHUNKS, IDX_CHUNK), jnp.int32),
            pltpu.VMEM((WV, D_MSG, LANES), jnp.float32),
            pltpu.VMEM((WV, D_MSG, LANES), jnp.float32),
            pltpu.VMEM((B_PER_W * D_MSG,), jnp.float32),
            pltpu.VMEM((B_PER_S,), jnp.float32),
            pltpu.VMEM((B_PER_S,), jnp.float32),
            pltpu.VMEM((L,), jnp.float32),
            pltpu.VMEM((NS, L), jnp.float32),
            pltpu.VMEM_SHARED((NS, L), jnp.float32),
            pltpu.SemaphoreType.DMA,
            pltpu.SemaphoreType.DMA,
            pltpu.SemaphoreType.DMA,
            pltpu.SemaphoreType.DMA,
        ],
    )(_sc_kernel_body)
    out_flat, ts = kfn(node_ids, prev_ts, msg_t, msg_ts)
    return jnp.reshape(out_flat, (BATCH, D_MSG)), ts


def kernel(node_ids, prev_ts, msg_store, msg_ts):
    return _last_message_gather(node_ids, prev_ts, msg_store, msg_ts)
